# baseline (speedup 1.0000x reference)
import functools
import math

import jax
import jax.numpy as jnp
from jax import lax
from jax.experimental import pallas as pl
from jax.experimental.pallas import tpu as pltpu
from jax.experimental.pallas import tpu_sc as plsc

VOCAB = 1000000
DIM = 64
ROWS = 4096
COLS = 200
B = ROWS * COLS
SCALE = math.sqrt(DIM)

NC = 2
NS = 16
NW = NC * NS
B_PER_W = B // NW
CHUNK = 800
N_CHUNKS = B_PER_W // CHUNK


def _emb_kernel(table_hbm, idx_hbm, out_hbm, idx_v, buf, sem):
    wid = lax.axis_index("s") * NC + lax.axis_index("c")
    base = wid * B_PER_W
    pltpu.sync_copy(idx_hbm.at[pl.ds(base, B_PER_W)], idx_v)

    def chunk_body(k, _):
        start = base + k * CHUNK
        pltpu.async_copy(
            table_hbm.at[idx_v.at[pl.ds(k * CHUNK, CHUNK)]], buf, sem
        ).wait()

        def scale_row(r, _):
            for u in range(DIM // 16):
                sl = pl.ds(u * 16, 16)
                buf[r, sl] = buf[r, sl] * SCALE
            return 0

        lax.fori_loop(0, CHUNK, scale_row, 0, unroll=2)

        pltpu.sync_copy(buf, out_hbm.at[pl.ds(start, CHUNK)])
        return 0

    lax.fori_loop(0, N_CHUNKS, chunk_body, 0)


@jax.jit
def _emb_call(idx_flat, table):
    mesh = plsc.VectorSubcoreMesh(core_axis_name="c", subcore_axis_name="s")
    kfn = pl.kernel(
        _emb_kernel,
        out_type=jax.ShapeDtypeStruct((B, DIM), jnp.float32),
        mesh=mesh,
        scratch_types=[
            pltpu.VMEM((B_PER_W,), jnp.int32),
            pltpu.VMEM((CHUNK, DIM), jnp.float32),
            pltpu.SemaphoreType.DMA,
        ],
        compiler_params=pltpu.CompilerParams(use_tc_tiling_on_sc=False),
    )
    return kfn(table, idx_flat)


def kernel(token_ids, table):
    idx_flat = token_ids.reshape(-1).astype(jnp.int32)
    out = _emb_call(idx_flat, table)
    return out.reshape(ROWS, COLS, DIM)


# speedup vs baseline: 1.0531x; 1.0531x over previous
import math

import jax
import jax.numpy as jnp
from jax import lax
from jax.experimental import pallas as pl
from jax.experimental.pallas import tpu as pltpu
from jax.experimental.pallas import tpu_sc as plsc

VOCAB = 1000000
DIM = 64
ROWS = 4096
COLS = 200
B = ROWS * COLS
SCALE = math.sqrt(DIM)

NC = 2
NS = 16
NW = NC * NS
B_PER_W = B // NW
CHUNK = 800
N_CHUNKS = B_PER_W // CHUNK


def _emb_kernel(table_hbm, idx_hbm, out_hbm, idx_v, buf0, buf1, gsem0,
                gsem1, wsem0, wsem1):
    wid = lax.axis_index("s") * NC + lax.axis_index("c")
    base = wid * B_PER_W
    pltpu.sync_copy(idx_hbm.at[pl.ds(base, B_PER_W)], idx_v)

    bufs = (buf0, buf1)
    gsems = (gsem0, gsem1)
    wsems = (wsem0, wsem1)

    def gather(k, b):
        pltpu.make_async_copy(
            table_hbm.at[idx_v.at[pl.ds(k * CHUNK, CHUNK)]], bufs[b],
            gsems[b],
        ).start()

    def scale(b):
        buf = bufs[b]

        @plsc.parallel_loop(0, CHUNK, unroll=4)
        def _(r):
            for u in range(DIM // 16):
                sl = pl.ds(u * 16, 16)
                buf[r, sl] = buf[r, sl] * SCALE

    gather(0, 0)
    for k in range(N_CHUNKS):
        b = k % 2
        pltpu.make_async_copy(
            table_hbm.at[idx_v.at[pl.ds(k * CHUNK, CHUNK)]], bufs[b],
            gsems[b],
        ).wait()
        if k + 1 < N_CHUNKS:
            if k >= 1:
                pltpu.make_async_copy(
                    bufs[1 - b],
                    out_hbm.at[pl.ds(base + (k - 1) * CHUNK, CHUNK)],
                    wsems[1 - b],
                ).wait()
            gather(k + 1, 1 - b)
        scale(b)
        pltpu.make_async_copy(
            bufs[b], out_hbm.at[pl.ds(base + k * CHUNK, CHUNK)], wsems[b]
        ).start()
    for k in (N_CHUNKS - 2, N_CHUNKS - 1):
        b = k % 2
        pltpu.make_async_copy(
            bufs[b], out_hbm.at[pl.ds(base + k * CHUNK, CHUNK)], wsems[b]
        ).wait()


@jax.jit
def _emb_call(idx_flat, table):
    mesh = plsc.VectorSubcoreMesh(core_axis_name="c", subcore_axis_name="s")
    kfn = pl.kernel(
        _emb_kernel,
        out_type=jax.ShapeDtypeStruct((B, DIM), jnp.float32),
        mesh=mesh,
        scratch_types=[
            pltpu.VMEM((B_PER_W,), jnp.int32),
            pltpu.VMEM((CHUNK, DIM), jnp.float32),
            pltpu.VMEM((CHUNK, DIM), jnp.float32),
            pltpu.SemaphoreType.DMA,
            pltpu.SemaphoreType.DMA,
            pltpu.SemaphoreType.DMA,
            pltpu.SemaphoreType.DMA,
        ],
        compiler_params=pltpu.CompilerParams(use_tc_tiling_on_sc=False),
    )
    return kfn(table, idx_flat)


def kernel(token_ids, table):
    idx_flat = token_ids.reshape(-1).astype(jnp.int32)
    out = _emb_call(idx_flat, table)
    return out.reshape(ROWS, COLS, DIM)


# speedup vs baseline: 1.0800x; 1.0255x over previous
import math

import jax
import jax.numpy as jnp
from jax import lax
from jax.experimental import pallas as pl
from jax.experimental.pallas import tpu as pltpu
from jax.experimental.pallas import tpu_sc as plsc

VOCAB = 1000000
DIM = 64
ROWS = 4096
COLS = 200
B = ROWS * COLS
SCALE = math.sqrt(DIM)

NC = 2
NS = 16
NW = NC * NS

RCHUNK = 200
N_RCHUNKS = VOCAB // RCHUNK
RCH_PER_W = 158

B_PER_W = B // NW
CHUNK = 160
N_CHUNKS = B_PER_W // CHUNK


def _repack_kernel(table_hbm, wide_hbm, bufa0, bufa1, bufw0, bufw1, rsem0,
                   rsem1, wsem0, wsem1):
    wid = lax.axis_index("s") * NC + lax.axis_index("c")
    bufas = (bufa0, bufa1)
    bufws = (bufw0, bufw1)
    rsems = (rsem0, rsem1)
    wsems = (wsem0, wsem1)

    def read(c, b):
        pltpu.make_async_copy(
            table_hbm.at[pl.ds(c * RCHUNK, RCHUNK)], bufas[b], rsems[b]
        ).start()

    def read_wait(c, b):
        pltpu.make_async_copy(
            table_hbm.at[pl.ds(c * RCHUNK, RCHUNK)], bufas[b], rsems[b]
        ).wait()

    def write(c, b):
        pltpu.make_async_copy(
            bufws[b], wide_hbm.at[pl.ds(c * RCHUNK, RCHUNK)], wsems[b]
        ).start()

    def write_wait(c, b):
        pltpu.make_async_copy(
            bufws[b], wide_hbm.at[pl.ds(c * RCHUNK, RCHUNK)], wsems[b]
        ).wait()

    def vcopy(b):
        src = bufas[b]
        dst = bufws[b]

        @plsc.parallel_loop(0, RCHUNK, unroll=4)
        def _(r):
            for u in range(DIM // 16):
                sl = pl.ds(u * 16, 16)
                dst[r, sl] = src[r, sl]

    read(wid, 0)

    def body(g, _):
        for b in range(2):
            j = 2 * g + b
            c = wid + j * NW

            @pl.when(c < N_RCHUNKS)
            def _():
                read_wait(c, b)

                @pl.when(c + NW < N_RCHUNKS)
                def _():
                    read(c + NW, 1 - b)

                @pl.when(j >= 2)
                def _():
                    write_wait(c - 2 * NW, b)

                vcopy(b)
                write(c, b)

        return 0

    lax.fori_loop(0, RCH_PER_W // 2, body, 0)

    def drain(g, _):
        for b in range(2):
            c = wid + (2 * g + b) * NW

            @pl.when((c >= N_RCHUNKS - 2 * NW) & (c < N_RCHUNKS))
            def _():
                write_wait(c, b)

        return 0

    lax.fori_loop(0, RCH_PER_W // 2, drain, 0)


def _gather_kernel(wide_hbm, idx_hbm, out_hbm, idx_v, buf0, buf1, pk0, pk1,
                   gsem0, gsem1, wsem0, wsem1):
    wid = lax.axis_index("s") * NC + lax.axis_index("c")
    base = wid * B_PER_W
    pltpu.sync_copy(idx_hbm.at[pl.ds(base, B_PER_W)], idx_v)

    bufs = (buf0, buf1)
    pks = (pk0, pk1)
    gsems = (gsem0, gsem1)
    wsems = (wsem0, wsem1)

    def gather(k, b):
        pltpu.make_async_copy(
            wide_hbm.at[idx_v.at[pl.ds(k * CHUNK, CHUNK)]], bufs[b],
            gsems[b],
        ).start()

    def gather_wait(k, b):
        pltpu.make_async_copy(
            wide_hbm.at[idx_v.at[pl.ds(k * CHUNK, CHUNK)]], bufs[b],
            gsems[b],
        ).wait()

    def scale_pack(b):
        buf = bufs[b]
        pk = pks[b]

        @plsc.parallel_loop(0, CHUNK, unroll=4)
        def _(r):
            for u in range(DIM // 16):
                sl = pl.ds(u * 16, 16)
                pk[r, sl] = buf[r, sl] * SCALE

    def write(k, b):
        pltpu.make_async_copy(
            pks[b], out_hbm.at[pl.ds(base + k * CHUNK, CHUNK)], wsems[b]
        ).start()

    def write_wait(k, b):
        pltpu.make_async_copy(
            pks[b], out_hbm.at[pl.ds(base + k * CHUNK, CHUNK)], wsems[b]
        ).wait()

    gather(0, 0)

    def body(g, _):
        for b in range(2):
            k = 2 * g + b
            gather_wait(k, b)

            @pl.when(k + 1 < N_CHUNKS)
            def _():
                gather(k + 1, 1 - b)

            @pl.when(k >= 2)
            def _():
                write_wait(k - 2, b)

            scale_pack(b)
            write(k, b)
        return 0

    lax.fori_loop(0, N_CHUNKS // 2, body, 0)
    write_wait(N_CHUNKS - 2, 0 if (N_CHUNKS - 2) % 2 == 0 else 1)
    write_wait(N_CHUNKS - 1, 0 if (N_CHUNKS - 1) % 2 == 0 else 1)


@jax.jit
def _emb_call(idx_flat, table):
    mesh = plsc.VectorSubcoreMesh(core_axis_name="c", subcore_axis_name="s")
    repack = pl.kernel(
        _repack_kernel,
        out_type=jax.ShapeDtypeStruct((VOCAB, 128), jnp.float32),
        mesh=mesh,
        scratch_types=[
            pltpu.VMEM((RCHUNK, DIM), jnp.float32),
            pltpu.VMEM((RCHUNK, DIM), jnp.float32),
            pltpu.VMEM((RCHUNK, 128), jnp.float32),
            pltpu.VMEM((RCHUNK, 128), jnp.float32),
            pltpu.SemaphoreType.DMA,
            pltpu.SemaphoreType.DMA,
            pltpu.SemaphoreType.DMA,
            pltpu.SemaphoreType.DMA,
        ],
    )
    wide = repack(table)
    gather = pl.kernel(
        _gather_kernel,
        out_type=jax.ShapeDtypeStruct((B, DIM), jnp.float32),
        mesh=mesh,
        scratch_types=[
            pltpu.VMEM((B_PER_W,), jnp.int32),
            pltpu.VMEM((CHUNK, 128), jnp.float32),
            pltpu.VMEM((CHUNK, 128), jnp.float32),
            pltpu.VMEM((CHUNK, DIM), jnp.float32),
            pltpu.VMEM((CHUNK, DIM), jnp.float32),
            pltpu.SemaphoreType.DMA,
            pltpu.SemaphoreType.DMA,
            pltpu.SemaphoreType.DMA,
            pltpu.SemaphoreType.DMA,
        ],
    )
    return gather(wide, idx_flat)


def kernel(token_ids, table):
    idx_flat = token_ids.reshape(-1).astype(jnp.int32)
    out = _emb_call(idx_flat, table)
    return out.reshape(ROWS, COLS, DIM)
